# Initial kernel scaffold; baseline (speedup 1.0000x reference)
import jax
import jax.numpy as jnp
from jax.experimental import pallas as pl


def kernel(hidden_states, W_gate, Wg, Wu, Wd):
    raise NotImplementedError("write your pallas kernel here")


# speedup vs baseline: 1.6932x; 1.6932x over previous
import functools

import jax
import jax.numpy as jnp
from jax.experimental import pallas as pl
from jax.experimental.pallas import tpu as pltpu

_HIDDEN = 1024
_E = 8
_DFF = 512
_BT = 512


def _router_kernel(x_ref, wg_ref, logits_ref, combine_ref):
    x = x_ref[...]
    logits = jax.lax.dot_general(
        x, wg_ref[...], (((1,), (0,)), ((), ())),
        preferred_element_type=jnp.float32)
    logits_ref[...] = logits
    m = jnp.max(logits, axis=1, keepdims=True)
    ex = jnp.exp(logits - m)
    s = ex / jnp.sum(ex, axis=1, keepdims=True)
    v1 = jnp.max(s, axis=1, keepdims=True)
    rest = jnp.where(s == v1, -jnp.inf, s)
    v2 = jnp.max(rest, axis=1, keepdims=True)
    sel = s >= v2
    combine_ref[...] = jnp.where(sel, s, 0.0) / (v1 + v2)


def _expert_kernel(x_ref, wg_ref, wu_ref, wd_ref, comb_ref, out_ref):
    e = pl.program_id(1)
    x = x_ref[...]
    g = jax.lax.dot_general(
        x, wg_ref[0], (((1,), (0,)), ((), ())),
        preferred_element_type=jnp.float32)
    u = jax.lax.dot_general(
        x, wu_ref[0], (((1,), (0,)), ((), ())),
        preferred_element_type=jnp.float32)
    h = (g * jax.lax.logistic(g) * u).astype(jnp.bfloat16)
    y = jax.lax.dot_general(
        h, wd_ref[0], (((1,), (0,)), ((), ())),
        preferred_element_type=jnp.float32)
    comb = comb_ref[...]
    lane = jax.lax.broadcasted_iota(jnp.int32, comb.shape, 1)
    w = jnp.sum(jnp.where(lane == e, comb, 0.0), axis=1, keepdims=True)
    contrib = w * y

    @pl.when(e == 0)
    def _init():
        out_ref[...] = contrib

    @pl.when(e != 0)
    def _acc():
        out_ref[...] += contrib


@jax.jit
def kernel(hidden_states, W_gate, Wg, Wu, Wd):
    b, s, d = hidden_states.shape
    x = hidden_states.reshape(-1, d)
    T = x.shape[0]

    logits, combine = pl.pallas_call(
        _router_kernel,
        out_shape=(
            jax.ShapeDtypeStruct((T, _E), jnp.float32),
            jax.ShapeDtypeStruct((T, _E), jnp.float32),
        ),
    )(x, W_gate)

    xb = x.astype(jnp.bfloat16)
    wgb = Wg.astype(jnp.bfloat16)
    wub = Wu.astype(jnp.bfloat16)
    wdb = Wd.astype(jnp.bfloat16)

    nblk = T // _BT
    out = pl.pallas_call(
        _expert_kernel,
        grid=(nblk, _E),
        in_specs=[
            pl.BlockSpec((_BT, _HIDDEN), lambda i, e: (i, 0)),
            pl.BlockSpec((1, _HIDDEN, _DFF), lambda i, e: (e, 0, 0)),
            pl.BlockSpec((1, _HIDDEN, _DFF), lambda i, e: (e, 0, 0)),
            pl.BlockSpec((1, _DFF, _HIDDEN), lambda i, e: (e, 0, 0)),
            pl.BlockSpec((_BT, _E), lambda i, e: (i, 0)),
        ],
        out_specs=pl.BlockSpec((_BT, _HIDDEN), lambda i, e: (i, 0)),
        out_shape=jax.ShapeDtypeStruct((T, _HIDDEN), jnp.float32),
        compiler_params=pltpu.CompilerParams(
            dimension_semantics=("parallel", "arbitrary")),
    )(xb, wgb, wub, wdb, combine)

    return out.reshape(b, s, d), logits


# speedup vs baseline: 1.7350x; 1.0247x over previous
import functools

import jax
import jax.numpy as jnp
from jax.experimental import pallas as pl
from jax.experimental.pallas import tpu as pltpu

_HIDDEN = 1024
_E = 8
_DFF = 512
_BT = 512


def _router_kernel(x_ref, wg_ref, logits_ref, combine_ref):
    x = x_ref[...]
    logits = jax.lax.dot_general(
        x, wg_ref[...], (((1,), (0,)), ((), ())),
        preferred_element_type=jnp.float32)
    logits_ref[...] = logits
    m = jnp.max(logits, axis=1, keepdims=True)
    ex = jnp.exp(logits - m)
    s = ex / jnp.sum(ex, axis=1, keepdims=True)
    v1 = jnp.max(s, axis=1, keepdims=True)
    rest = jnp.where(s == v1, -jnp.inf, s)
    v2 = jnp.max(rest, axis=1, keepdims=True)
    sel = s >= v2
    combine_ref[...] = jnp.where(sel, s, 0.0) / (v1 + v2)


def _expert_kernel(x_ref, wg_ref, wu_ref, wd_ref, comb_ref, out_ref):
    e = pl.program_id(1)
    x = x_ref[...]
    g = jax.lax.dot_general(
        x, wg_ref[0], (((1,), (0,)), ((), ())),
        preferred_element_type=jnp.float32)
    u = jax.lax.dot_general(
        x, wu_ref[0], (((1,), (0,)), ((), ())),
        preferred_element_type=jnp.float32)
    h = (g * jax.lax.logistic(g) * u).astype(jnp.bfloat16)
    y = jax.lax.dot_general(
        h, wd_ref[0], (((1,), (0,)), ((), ())),
        preferred_element_type=jnp.float32)
    comb = comb_ref[...]
    lane = jax.lax.broadcasted_iota(jnp.int32, comb.shape, 1)
    w = jnp.sum(jnp.where(lane == e, comb, 0.0), axis=1, keepdims=True)
    contrib = w * y

    @pl.when(e == 0)
    def _init():
        out_ref[...] = contrib

    @pl.when(e != 0)
    def _acc():
        out_ref[...] += contrib


@jax.jit
def kernel(hidden_states, W_gate, Wg, Wu, Wd):
    b, s, d = hidden_states.shape
    x = hidden_states.reshape(-1, d)
    T = x.shape[0]

    logits, combine = pl.pallas_call(
        _router_kernel,
        out_shape=(
            jax.ShapeDtypeStruct((T, _E), jnp.float32),
            jax.ShapeDtypeStruct((T, _E), jnp.float32),
        ),
    )(x, W_gate)

    xb = x.astype(jnp.bfloat16)
    wgb = Wg.astype(jnp.bfloat16)
    wub = Wu.astype(jnp.bfloat16)
    wdb = Wd.astype(jnp.bfloat16)

    out = pl.pallas_call(
        _expert_kernel,
        grid=(1, _E),
        in_specs=[
            pl.BlockSpec((T, _HIDDEN), lambda i, e: (i, 0)),
            pl.BlockSpec((1, _HIDDEN, _DFF), lambda i, e: (e, 0, 0)),
            pl.BlockSpec((1, _HIDDEN, _DFF), lambda i, e: (e, 0, 0)),
            pl.BlockSpec((1, _DFF, _HIDDEN), lambda i, e: (e, 0, 0)),
            pl.BlockSpec((T, _E), lambda i, e: (i, 0)),
        ],
        out_specs=pl.BlockSpec((T, _HIDDEN), lambda i, e: (i, 0)),
        out_shape=jax.ShapeDtypeStruct((T, _HIDDEN), jnp.float32),
        compiler_params=pltpu.CompilerParams(
            dimension_semantics=("arbitrary", "arbitrary")),
    )(xb, wgb, wub, wdb, combine)

    return out.reshape(b, s, d), logits
